# baseline (speedup 1.0000x reference)
import functools

import jax
import jax.numpy as jnp
from jax import lax
from jax.experimental import pallas as pl
from jax.experimental.pallas import tpu as pltpu
from jax.experimental.pallas import tpu_sc as plsc

N = 10000
E = 320000
D = 128
H = 128
ALPHA = 0.01

NC = 2
NS = 16
HH = H // NC
K = 128
NCH = 160
EPT = NCH * K
EPAD = NS * EPT
LANES = 16
RPT = 624
TAIL = N - RPT * NS


def _mm_body(d_ref, n_ref, w_ref, o_ref):
    feat = d_ref[...] + ALPHA * n_ref[...]
    y = jnp.dot(feat, w_ref[...], preferred_element_type=jnp.float32)
    o_ref[0] = y[:, :HH]
    o_ref[1] = y[:, HH:]


def _matmul(data, noise, W):
    blk = 1000
    return pl.pallas_call(
        _mm_body,
        grid=(N // blk,),
        in_specs=[
            pl.BlockSpec((blk, D), lambda i: (i, 0)),
            pl.BlockSpec((blk, D), lambda i: (i, 0)),
            pl.BlockSpec((D, H), lambda i: (0, 0)),
        ],
        out_specs=pl.BlockSpec((NC, blk, HH), lambda i: (0, i, 0)),
        out_shape=jax.ShapeDtypeStruct((NC, N, HH), jnp.float32),
    )(data, noise, W)


NBUF = 2


def _spmm_body(xs_hbm, rc_hbm, val_hbm, out_hbm,
               rc_v, vbufs, bufs, cols, rowbs, xsp, acc, gsems, vsems, ssems):
    cid = lax.axis_index("c")
    sid = lax.axis_index("s")
    x_hbm = xs_hbm.at[cid]
    buf0 = bufs[0]

    pltpu.sync_copy(rc_hbm.at[sid], rc_v)

    def _unpack_col(j, cb):
        for g in range(K // LANES):
            sl = pl.ds(g * LANES, LANES)
            cb[sl] = jnp.bitwise_and(rc_v[j, sl], 0xFFFF)

    def _unpack_row(j, rb):
        for g in range(K // LANES):
            sl = pl.ds(g * LANES, LANES)
            rb[sl] = jnp.right_shift(rc_v[j, sl], 16)

    zeros16 = jnp.zeros((LANES,), jnp.float32)

    def _zrow(e, carry):
        for u in range(HH // LANES):
            buf0[e, pl.ds(u * LANES, LANES)] = zeros16
            bufs[1][e, pl.ds(u * LANES, LANES)] = zeros16
        return carry

    lax.fori_loop(0, K, _zrow, 0)

    base_r = pl.multiple_of(sid * RPT, 8)
    rem = RPT % K
    for q in range(RPT // K):
        pltpu.sync_copy(buf0, acc.at[pl.ds(base_r + q * K, K)])
    if rem:
        pltpu.sync_copy(buf0.at[pl.ds(0, rem)],
                        acc.at[pl.ds(base_r + (RPT // K) * K, rem)])

    @pl.when(sid == NS - 1)
    def _zero_tail():
        pltpu.sync_copy(buf0.at[pl.ds(0, TAIL)], acc.at[pl.ds(RPT * NS, TAIL)])

    pltpu.sync_copy(x_hbm.at[pl.ds(base_r, RPT)], xsp.at[pl.ds(base_r, RPT)])

    @pl.when(sid == NS - 1)
    def _stage_tail():
        sl = pl.ds(RPT * NS, TAIL)
        pltpu.sync_copy(x_hbm.at[sl], xsp.at[sl])

    plsc.subcore_barrier()

    def _scale(j, buf, vbuf):
        for g in range(K // LANES):
            vv = vbuf[pl.ds(g * LANES, LANES)]
            for e16 in range(LANES):
                v = vv[e16]
                e = g * LANES + e16
                for u in range(HH // LANES):
                    sl = pl.ds(u * LANES, LANES)
                    buf[e, sl] = buf[e, sl] * v

    dummy = x_hbm.at[pl.ds(0, K)]
    vdummy = val_hbm.at[0, 0]

    for g in range(K // LANES):
        rowbs[1][pl.ds(g * LANES, LANES)] = jnp.zeros((LANES,), jnp.int32)
    pltpu.async_copy(bufs[1], acc.at[rowbs[1]], ssems[1], add=True)

    _unpack_col(0, cols[0])
    pltpu.async_copy(xsp.at[cols[0]], bufs[0], gsems[0])
    pltpu.async_copy(val_hbm.at[sid, 0], vbufs[0], vsems[0])

    def _ring(t, carry):
        for p in range(NBUF):
            q = 1 - p
            j = NBUF * t + p
            pltpu.make_async_copy(dummy, bufs[p], gsems[p]).wait()
            pltpu.make_async_copy(vdummy, vbufs[p], vsems[p]).wait()
            jn = jnp.minimum(j + 1, NCH - 1)
            pltpu.make_async_copy(dummy, bufs[q], ssems[q]).wait()
            _unpack_col(jn, cols[q])
            pltpu.async_copy(xsp.at[cols[q]], bufs[q], gsems[q])
            pltpu.async_copy(val_hbm.at[sid, jn], vbufs[q], vsems[q])
            _scale(j, bufs[p], vbufs[p])
            _unpack_row(j, rowbs[p])
            pltpu.async_copy(bufs[p], acc.at[rowbs[p]], ssems[p], add=True)
        return carry

    lax.fori_loop(0, NCH // NBUF, _ring, 0)
    pltpu.make_async_copy(dummy, bufs[0], gsems[0]).wait()
    pltpu.make_async_copy(vdummy, vbufs[0], vsems[0]).wait()
    pltpu.make_async_copy(dummy, bufs[1], ssems[1]).wait()
    plsc.subcore_barrier()

    for q in range(RPT // K):
        sl = pl.ds(base_r + q * K, K)
        pltpu.sync_copy(acc.at[sl], out_hbm.at[cid].at[sl])
    if rem:
        sl = pl.ds(base_r + (RPT // K) * K, rem)
        pltpu.sync_copy(acc.at[sl], out_hbm.at[cid].at[sl])

    @pl.when(sid == NS - 1)
    def _drain_tail():
        sl = pl.ds(RPT * NS, TAIL)
        pltpu.sync_copy(acc.at[sl], out_hbm.at[cid].at[sl])


@functools.cache
def _make_spmm():
    return pl.kernel(
        _spmm_body,
        out_type=jax.ShapeDtypeStruct((NC, N, HH), jnp.float32),
        mesh=plsc.VectorSubcoreMesh(core_axis_name="c", subcore_axis_name="s",
                                    num_cores=NC, num_subcores=NS),
        compiler_params=pltpu.CompilerParams(use_tc_tiling_on_sc=False),
        scratch_types=[
            pltpu.VMEM((NCH, K), jnp.int32),
            [pltpu.VMEM((K,), jnp.float32) for _ in range(NBUF)],
            [pltpu.VMEM((K, HH), jnp.float32) for _ in range(NBUF)],
            [pltpu.VMEM((K,), jnp.int32) for _ in range(NBUF)],
            [pltpu.VMEM((K,), jnp.int32) for _ in range(NBUF)],
            pltpu.VMEM_SHARED((N, HH), jnp.float32),
            pltpu.VMEM_SHARED((N, HH), jnp.float32),
            [pltpu.SemaphoreType.DMA for _ in range(NBUF)],
            [pltpu.SemaphoreType.DMA for _ in range(NBUF)],
            [pltpu.SemaphoreType.DMA for _ in range(NBUF)],
        ],
    )


def _elu_body(p_ref, o_ref):
    for c in range(NC):
        s = p_ref[c]
        o_ref[:, c * HH:(c + 1) * HH] = jnp.where(
            s > 0, s, jnp.exp(jnp.minimum(s, 0.0)) - 1.0)


def _elu_concat(partials):
    blk = 1000
    return pl.pallas_call(
        _elu_body,
        grid=(N // blk,),
        in_specs=[pl.BlockSpec((NC, blk, HH), lambda i: (0, i, 0))],
        out_specs=pl.BlockSpec((blk, H), lambda i: (i, 0)),
        out_shape=jax.ShapeDtypeStruct((N, H), jnp.float32),
    )(partials)


def kernel(data, adj_indices, adj_values, W):
    noise = jax.random.normal(jax.random.key(42), (N, D), dtype=jnp.float32)
    xs = _matmul(data, noise, W)
    pad = EPAD - E
    rc = jnp.left_shift(adj_indices[0], 16) | adj_indices[1]
    rc = jnp.pad(rc, (0, pad)).reshape(NS, NCH, K)
    val = jnp.pad(adj_values, (0, pad)).reshape(NS, NCH, K)
    partials = _make_spmm()(xs, rc, val)
    return _elu_concat(partials)


# speedup vs baseline: 1.0393x; 1.0393x over previous
import functools

import jax
import jax.numpy as jnp
from jax import lax
from jax.experimental import pallas as pl
from jax.experimental.pallas import tpu as pltpu
from jax.experimental.pallas import tpu_sc as plsc

N = 10000
E = 320000
D = 128
H = 128
ALPHA = 0.01

NC = 2
NS = 16
HH = H // NC
K = 128
NCH = 160
EPT = NCH * K
EPAD = NS * EPT
LANES = 16
RPT = 624
TAIL = N - RPT * NS


def _mm_body(d_ref, n_ref, w_ref, o_ref):
    feat = d_ref[...] + ALPHA * n_ref[...]
    y = jnp.dot(feat, w_ref[...], preferred_element_type=jnp.float32)
    y = y.astype(jnp.bfloat16)
    o_ref[0] = y[:, :HH]
    o_ref[1] = y[:, HH:]


def _matmul(data, noise, W):
    blk = 1000
    return pl.pallas_call(
        _mm_body,
        grid=(N // blk,),
        in_specs=[
            pl.BlockSpec((blk, D), lambda i: (i, 0)),
            pl.BlockSpec((blk, D), lambda i: (i, 0)),
            pl.BlockSpec((D, H), lambda i: (0, 0)),
        ],
        out_specs=pl.BlockSpec((NC, blk, HH), lambda i: (0, i, 0)),
        out_shape=jax.ShapeDtypeStruct((NC, N, HH), jnp.bfloat16),
    )(data, noise, W)


NBUF = 2


def _spmm_body(xs_hbm, rc_hbm, val_hbm, out_hbm,
               rc_v, vbufs, gbufs, sbufs, cols, rowbs, xsp, acc,
               gsems, vsems, ssems):
    cid = lax.axis_index("c")
    sid = lax.axis_index("s")
    x_hbm = xs_hbm.at[cid]

    pltpu.sync_copy(rc_hbm.at[sid], rc_v)

    def _unpack_col(j, cb):
        for g in range(K // LANES):
            sl = pl.ds(g * LANES, LANES)
            cb[sl] = jnp.bitwise_and(rc_v[j, sl], 0xFFFF)

    def _unpack_row(j, rb):
        for g in range(K // LANES):
            sl = pl.ds(g * LANES, LANES)
            rb[sl] = jnp.right_shift(rc_v[j, sl], 16)

    zeros16 = jnp.zeros((LANES,), jnp.float32)

    def _zrow(e, carry):
        for u in range(HH // LANES):
            sbufs[0][e, pl.ds(u * LANES, LANES)] = zeros16
            sbufs[1][e, pl.ds(u * LANES, LANES)] = zeros16
        return carry

    lax.fori_loop(0, K, _zrow, 0)

    base_r = pl.multiple_of(sid * RPT, 8)
    rem = RPT % K
    for q in range(RPT // K):
        pltpu.sync_copy(sbufs[0], acc.at[pl.ds(base_r + q * K, K)])
    if rem:
        pltpu.sync_copy(sbufs[0].at[pl.ds(0, rem)],
                        acc.at[pl.ds(base_r + (RPT // K) * K, rem)])

    @pl.when(sid == NS - 1)
    def _zero_tail():
        pltpu.sync_copy(sbufs[0].at[pl.ds(0, TAIL)],
                        acc.at[pl.ds(RPT * NS, TAIL)])

    pltpu.sync_copy(x_hbm.at[pl.ds(base_r, RPT)], xsp.at[pl.ds(base_r, RPT)])

    @pl.when(sid == NS - 1)
    def _stage_tail():
        sl = pl.ds(RPT * NS, TAIL)
        pltpu.sync_copy(x_hbm.at[sl], xsp.at[sl])

    plsc.subcore_barrier()

    def _scale(j, gbuf, sbuf, vbuf):
        for g in range(K // LANES):
            vv = vbuf[pl.ds(g * LANES, LANES)]
            for e16 in range(LANES):
                v = vv[e16]
                e = g * LANES + e16
                for u in range(HH // 32):
                    gb = gbuf[e, pl.ds(u * 32, 32)]
                    a, b = plsc.unpack(gb, format=plsc.PackFormat.INTERLEAVED,
                                       preferred_element_type=jnp.float32)
                    sbuf[e, pl.ds(u * 32, LANES)] = a * v
                    sbuf[e, pl.ds(u * 32 + LANES, LANES)] = b * v

    dummy = x_hbm.at[pl.ds(0, K)]
    vdummy = val_hbm.at[0, 0]
    sdummy = out_hbm.at[cid].at[pl.ds(0, K)]

    for g in range(K // LANES):
        rowbs[1][pl.ds(g * LANES, LANES)] = jnp.zeros((LANES,), jnp.int32)
    pltpu.async_copy(sbufs[0], acc.at[rowbs[1]], ssems[0], add=True)
    pltpu.async_copy(sbufs[1], acc.at[rowbs[1]], ssems[1], add=True)

    _unpack_col(0, cols[0])
    pltpu.async_copy(xsp.at[cols[0]], gbufs[0], gsems[0])
    pltpu.async_copy(val_hbm.at[sid, 0], vbufs[0], vsems[0])

    def _ring(t, carry):
        for p in range(NBUF):
            q = 1 - p
            j = NBUF * t + p
            pltpu.make_async_copy(dummy, gbufs[p], gsems[p]).wait()
            pltpu.make_async_copy(vdummy, vbufs[p], vsems[p]).wait()
            jn = jnp.minimum(j + 1, NCH - 1)
            _unpack_col(jn, cols[q])
            pltpu.async_copy(xsp.at[cols[q]], gbufs[q], gsems[q])
            pltpu.async_copy(val_hbm.at[sid, jn], vbufs[q], vsems[q])
            pltpu.make_async_copy(sdummy, sbufs[p], ssems[p]).wait()
            _scale(j, gbufs[p], sbufs[p], vbufs[p])
            _unpack_row(j, rowbs[p])
            pltpu.async_copy(sbufs[p], acc.at[rowbs[p]], ssems[p], add=True)
        return carry

    lax.fori_loop(0, NCH // NBUF, _ring, 0)
    pltpu.make_async_copy(dummy, gbufs[0], gsems[0]).wait()
    pltpu.make_async_copy(vdummy, vbufs[0], vsems[0]).wait()
    pltpu.make_async_copy(sdummy, sbufs[0], ssems[0]).wait()
    pltpu.make_async_copy(sdummy, sbufs[1], ssems[1]).wait()
    plsc.subcore_barrier()

    for q in range(RPT // K):
        sl = pl.ds(base_r + q * K, K)
        pltpu.sync_copy(acc.at[sl], out_hbm.at[cid].at[sl])
    if rem:
        sl = pl.ds(base_r + (RPT // K) * K, rem)
        pltpu.sync_copy(acc.at[sl], out_hbm.at[cid].at[sl])

    @pl.when(sid == NS - 1)
    def _drain_tail():
        sl = pl.ds(RPT * NS, TAIL)
        pltpu.sync_copy(acc.at[sl], out_hbm.at[cid].at[sl])


@functools.cache
def _make_spmm():
    return pl.kernel(
        _spmm_body,
        out_type=jax.ShapeDtypeStruct((NC, N, HH), jnp.float32),
        mesh=plsc.VectorSubcoreMesh(core_axis_name="c", subcore_axis_name="s",
                                    num_cores=NC, num_subcores=NS),
        compiler_params=pltpu.CompilerParams(use_tc_tiling_on_sc=False,
                                             needs_layout_passes=False),
        scratch_types=[
            pltpu.VMEM((NCH, K), jnp.int32),
            [pltpu.VMEM((K,), jnp.float32) for _ in range(NBUF)],
            [pltpu.VMEM((K, HH), jnp.bfloat16) for _ in range(NBUF)],
            [pltpu.VMEM((K, HH), jnp.float32) for _ in range(NBUF)],
            [pltpu.VMEM((K,), jnp.int32) for _ in range(NBUF)],
            [pltpu.VMEM((K,), jnp.int32) for _ in range(NBUF)],
            pltpu.VMEM_SHARED((N, HH), jnp.bfloat16),
            pltpu.VMEM_SHARED((N, HH), jnp.float32),
            [pltpu.SemaphoreType.DMA for _ in range(NBUF)],
            [pltpu.SemaphoreType.DMA for _ in range(NBUF)],
            [pltpu.SemaphoreType.DMA for _ in range(NBUF)],
        ],
    )


def _elu_body(p_ref, o_ref):
    for c in range(NC):
        s = p_ref[c]
        o_ref[:, c * HH:(c + 1) * HH] = jnp.where(
            s > 0, s, jnp.exp(jnp.minimum(s, 0.0)) - 1.0)


def _elu_concat(partials):
    blk = 1000
    return pl.pallas_call(
        _elu_body,
        grid=(N // blk,),
        in_specs=[pl.BlockSpec((NC, blk, HH), lambda i: (0, i, 0))],
        out_specs=pl.BlockSpec((blk, H), lambda i: (i, 0)),
        out_shape=jax.ShapeDtypeStruct((N, H), jnp.float32),
    )(partials)


_WPERM = tuple(
    base + (i // 2 if i % 2 == 0 else 16 + i // 2)
    for base in range(0, H, 32)
    for i in range(32)
)


def kernel(data, adj_indices, adj_values, W):
    noise = jax.random.normal(jax.random.key(42), (N, D), dtype=jnp.float32)
    xs = _matmul(data, noise, W[:, jnp.array(_WPERM, dtype=jnp.int32)])
    pad = EPAD - E
    rc = jnp.left_shift(adj_indices[0], 16) | adj_indices[1]
    rc = jnp.pad(rc, (0, pad)).reshape(NS, NCH, K)
    val = jnp.pad(adj_values, (0, pad)).reshape(NS, NCH, K)
    partials = _make_spmm()(xs, rc, val)
    return _elu_concat(partials)
